# baseline (speedup 1.0000x reference)
import jax
import jax.numpy as jnp
from jax import lax
from jax.experimental import pallas as pl
from jax.experimental.pallas import tpu as pltpu
from jax.experimental.pallas import tpu_sc as plsc

N = 10000
D = 256
C = 6
MAXLEN = 8
IBIG = 2 ** 30

L = 16
NW = 32
RBIG = 320
RSML = 304
CH = 400
NCH = N // CH


def _iota():
    return lax.broadcasted_iota(jnp.int32, (L,), 0)


def _splat(v):
    return jnp.full((L,), v, jnp.int32)


def _sc_body(q_hbm, bm_hbm, rp_hbm, out_hbm, q_v, bm_v, rp_v, bma_v, s_v,
             qsem, bsem):
    wid = lax.axis_index("s") * 2 + lax.axis_index("c")
    big = wid < 17
    ng = jnp.where(big, RBIG // L, RSML // L)
    r0 = jnp.where(big, wid * RBIG, 17 * RBIG + (wid - 17) * RSML)
    rbase = jnp.minimum(r0, N - RBIG)
    loff = r0 - rbase

    cp_q = pltpu.async_copy(q_hbm.at[pl.ds(rbase, RBIG)], q_v, qsem)
    cps = []
    for c in range(C):
        cps.append(pltpu.async_copy(
            bm_hbm.at[c, pl.ds(rbase, RBIG), :], bm_v.at[c], bsem))
        cps.append(pltpu.async_copy(
            rp_hbm.at[c, pl.ds(rbase, RBIG), :], rp_v.at[c], bsem))

    iota = _iota()

    thresh = []
    for c in range(C):
        def cond(st):
            k, _, _, done = st
            return jnp.logical_and(k < NCH, jnp.logical_not(done))

        def chunk_body(st, c=c):
            k, cnt, th, done = st
            start = k * CH
            pltpu.sync_copy(bm_hbm.at[c, pl.ds(start, CH), :], bma_v)
            kth = 8 - cnt

            def g(j, acc):
                cum, best = acc
                rl = j * L + iota
                h = plsc.load_gather(bma_v, [rl, _splat(0)])
                for p in range(1, 4):
                    h = h + plsc.load_gather(bma_v, [rl, _splat(p)])
                hit01 = (h > 0).astype(jnp.int32)
                cumv = plsc.cumsum(hit01) + cum
                at_kth = jnp.logical_and(hit01 > 0, cumv == kth)
                pos = jnp.min(jnp.where(at_kth, start + rl, IBIG))
                return cum + jnp.sum(hit01), jnp.minimum(best, pos)

            total, best = lax.fori_loop(0, CH // L, g,
                                        (jnp.int32(0), jnp.int32(IBIG)))
            found = jnp.logical_and(jnp.logical_not(done),
                                    (cnt + total) >= MAXLEN)
            th = jnp.where(found, best, th)
            return k + 1, cnt + total, th, jnp.logical_or(done, found)

        _, _, th_c, _ = lax.while_loop(
            cond, chunk_body,
            (jnp.int32(0), jnp.int32(0), jnp.int32(IBIG), jnp.bool_(False)))
        thresh.append(th_c)

    for cp in cps:
        cp.wait()
    cp_q.wait()

    def group(j, _):
        rl = loff + j * L + iota
        rg = r0 + j * L + iota
        cntv = jnp.zeros((L,), jnp.float32)
        ssum = jnp.zeros((L,), jnp.float32)
        for c in range(C):
            h = plsc.load_gather(bm_v, [_splat(c), rl, _splat(0)])
            for p in range(1, 4):
                h = h + plsc.load_gather(bm_v, [_splat(c), rl, _splat(p)])
            hit = h > 0
            cntv = cntv + jnp.where(hit, 1.0, 0.0)
            rs = plsc.load_gather(rp_v, [_splat(c), rl, _splat(0)])
            for p in range(1, 8):
                rs = rs + plsc.load_gather(rp_v, [_splat(c), rl, _splat(p)])
            e = jnp.exp(rs * 0.25)
            attn = (e - 1.0) / (e + 1.0)
            sel = jnp.logical_and(hit, rg <= thresh[c])
            ssum = ssum + jnp.where(sel, 1.0 + attn, 0.0)
        s_v[pl.ds(L, L)] = ssum / jnp.maximum(cntv, 1.0)
        for i in range(L):
            scale = plsc.load_gather(s_v, [_splat(L + i)])
            row = loff + j * L + i
            for k2 in range(D // L):
                sl = pl.ds(k2 * L, L)
                q_v[row, sl] = q_v[row, sl] * scale
        return 0

    lax.fori_loop(0, ng, group, 0)

    @pl.when(big)
    def _():
        pltpu.sync_copy(q_v.at[pl.ds(0, RBIG)], out_hbm.at[pl.ds(r0, RBIG)])

    @pl.when(jnp.logical_not(big))
    def _():
        pltpu.sync_copy(q_v.at[pl.ds(loff, RSML)],
                        out_hbm.at[pl.ds(r0, RSML)])


def kernel(query, reference_points_cam, bev_mask):
    q = query[0]
    bm = bev_mask[:, 0]
    rp = reference_points_cam[:, 0].reshape(C, N, 8)

    mesh = plsc.VectorSubcoreMesh(core_axis_name="c", subcore_axis_name="s",
                                  num_cores=2, num_subcores=16)
    run = pl.kernel(
        _sc_body,
        out_type=jax.ShapeDtypeStruct((N, D), jnp.float32),
        mesh=mesh,
        compiler_params=pltpu.CompilerParams(needs_layout_passes=False,
                                             use_tc_tiling_on_sc=False),
        scratch_types=[
            pltpu.VMEM((RBIG, D), jnp.float32),
            pltpu.VMEM((C, RBIG, 4), jnp.int32),
            pltpu.VMEM((C, RBIG, 8), jnp.float32),
            pltpu.VMEM((CH, 4), jnp.int32),
            pltpu.VMEM((2 * L,), jnp.float32),
            pltpu.SemaphoreType.DMA,
            pltpu.SemaphoreType.DMA,
        ],
    )
    return run(q, bm, rp)[None]


# speedup vs baseline: 1.2662x; 1.2662x over previous
import jax
import jax.numpy as jnp
from jax import lax
from jax.experimental import pallas as pl
from jax.experimental.pallas import tpu as pltpu
from jax.experimental.pallas import tpu_sc as plsc

N = 10000
D = 256
C = 6
MAXLEN = 8
IBIG = 2 ** 30

L = 16
RBIG = 320
RSML = 304
CH = 400
NCH = N // CH


def _iota():
    return lax.broadcasted_iota(jnp.int32, (L,), 0)


def _splat(v):
    return jnp.full((L,), v, jnp.int32)


def _sc_body(bm_hbm, rp_hbm, s_hbm, bm_v, rp_v, bma_v, s_v, bsem):
    wid = lax.axis_index("s") * 2 + lax.axis_index("c")
    big = wid < 17
    ng = jnp.where(big, RBIG // L, RSML // L)
    r0 = jnp.where(big, wid * RBIG, 17 * RBIG + (wid - 17) * RSML)
    rbase = jnp.minimum(r0, N - RBIG)
    loff = r0 - rbase

    cps = []
    for c in range(C):
        cps.append(pltpu.async_copy(
            bm_hbm.at[c, pl.ds(rbase, RBIG), :], bm_v.at[c], bsem))
        cps.append(pltpu.async_copy(
            rp_hbm.at[c, pl.ds(rbase, RBIG), :], rp_v.at[c], bsem))

    iota = _iota()

    thresh = []
    for c in range(C):
        def cond(st):
            k, _, _, done = st
            return jnp.logical_and(k < NCH, jnp.logical_not(done))

        def chunk_body(st, c=c):
            k, cnt, th, done = st
            start = k * CH
            pltpu.sync_copy(bm_hbm.at[c, pl.ds(start, CH), :], bma_v)
            kth = MAXLEN - cnt

            def g(j, acc):
                cum, best = acc
                rl = j * L + iota
                h = plsc.load_gather(bma_v, [rl, _splat(0)])
                for p in range(1, 4):
                    h = h + plsc.load_gather(bma_v, [rl, _splat(p)])
                hit01 = (h > 0).astype(jnp.int32)
                cumv = plsc.cumsum(hit01) + cum
                at_kth = jnp.logical_and(hit01 > 0, cumv == kth)
                pos = jnp.min(jnp.where(at_kth, start + rl, IBIG))
                return cum + jnp.sum(hit01), jnp.minimum(best, pos)

            total, best = lax.fori_loop(0, CH // L, g,
                                        (jnp.int32(0), jnp.int32(IBIG)))
            found = jnp.logical_and(jnp.logical_not(done),
                                    (cnt + total) >= MAXLEN)
            th = jnp.where(found, best, th)
            return k + 1, cnt + total, th, jnp.logical_or(done, found)

        _, _, th_c, _ = lax.while_loop(
            cond, chunk_body,
            (jnp.int32(0), jnp.int32(0), jnp.int32(IBIG), jnp.bool_(False)))
        thresh.append(th_c)

    maxth = thresh[0]
    for c in range(1, C):
        maxth = jnp.maximum(maxth, thresh[c])

    for cp in cps:
        cp.wait()

    def group(j, _):
        zero = jnp.zeros((L,), jnp.float32)
        s_v[pl.ds(loff + j * L, L)] = zero

        @pl.when(r0 + j * L <= maxth)
        def _():
            rl = loff + j * L + iota
            rg = r0 + j * L + iota
            cntv = jnp.zeros((L,), jnp.float32)
            ssum = jnp.zeros((L,), jnp.float32)
            for c in range(C):
                h = plsc.load_gather(bm_v, [_splat(c), rl, _splat(0)])
                for p in range(1, 4):
                    h = h + plsc.load_gather(bm_v, [_splat(c), rl, _splat(p)])
                hit = h > 0
                cntv = cntv + jnp.where(hit, 1.0, 0.0)
                rs = plsc.load_gather(rp_v, [_splat(c), rl, _splat(0)])
                for p in range(1, 8):
                    rs = rs + plsc.load_gather(rp_v,
                                               [_splat(c), rl, _splat(p)])
                e = jnp.exp(rs * 0.25)
                attn = (e - 1.0) / (e + 1.0)
                sel = jnp.logical_and(hit, rg <= thresh[c])
                ssum = ssum + jnp.where(sel, 1.0 + attn, 0.0)
            s_v[pl.ds(loff + j * L, L)] = ssum / jnp.maximum(cntv, 1.0)
        return 0

    lax.fori_loop(0, ng, group, 0)

    @pl.when(big)
    def _():
        pltpu.sync_copy(s_v.at[pl.ds(0, RBIG)], s_hbm.at[pl.ds(r0, RBIG)])

    @pl.when(jnp.logical_not(big))
    def _():
        pltpu.sync_copy(s_v.at[pl.ds(loff, RSML)],
                        s_hbm.at[pl.ds(r0, RSML)])


def _tc_body(q_ref, s_ref, o_ref):
    o_ref[...] = q_ref[...] * s_ref[...].T


def kernel(query, reference_points_cam, bev_mask):
    q = query[0]
    bm = bev_mask[:, 0]
    rp = reference_points_cam[:, 0].reshape(C, N, 8)

    mesh = plsc.VectorSubcoreMesh(core_axis_name="c", subcore_axis_name="s",
                                  num_cores=2, num_subcores=16)
    s = pl.kernel(
        _sc_body,
        out_type=jax.ShapeDtypeStruct((N,), jnp.float32),
        mesh=mesh,
        compiler_params=pltpu.CompilerParams(needs_layout_passes=False,
                                             use_tc_tiling_on_sc=False),
        scratch_types=[
            pltpu.VMEM((C, RBIG, 4), jnp.int32),
            pltpu.VMEM((C, RBIG, 8), jnp.float32),
            pltpu.VMEM((CH, 4), jnp.int32),
            pltpu.VMEM((RBIG,), jnp.float32),
            pltpu.SemaphoreType.DMA,
        ],
    )(bm, rp)

    out = pl.pallas_call(
        _tc_body,
        out_shape=jax.ShapeDtypeStruct((N, D), jnp.float32),
    )(q, s.reshape(1, N))
    return out[None]


# speedup vs baseline: 10.1895x; 8.0472x over previous
import jax
import jax.numpy as jnp
from jax.experimental import pallas as pl
from jax.experimental.pallas import tpu as pltpu

N = 10000
D = 256
C = 6
MAXLEN = 8
BIG = 2 ** 30
BLK = 1000
NBLK = N // BLK


def _msda_body(q_hbm, bm_ref, rp_ref, o_ref, s_ref, flag_ref, qbuf, sem):
    b = pl.program_id(0)

    @pl.when(b == 0)
    def _():
        hits = bm_ref[0] + bm_ref[1] + bm_ref[2] + bm_ref[3]
        hit = hits > 0
        hit_f = hit.astype(jnp.float32)

        count = jnp.sum(hit_f, axis=0, keepdims=True)
        cnorm = 1.0 / jnp.maximum(count, 1.0)

        iota = jax.lax.broadcasted_iota(jnp.int32, hit.shape, 1)
        masked = jnp.where(hit, iota, BIG)
        thresh = None
        for _ in range(MAXLEN):
            thresh = jnp.min(masked, axis=1, keepdims=True)
            masked = jnp.where(masked == thresh, BIG, masked)
        sel = hit_f * (iota <= thresh).astype(jnp.float32)

        rsum = rp_ref[0]
        for p in range(1, 8):
            rsum = rsum + rp_ref[p]
        attn = jnp.tanh(rsum * 0.125)

        s = jnp.sum(sel * (1.0 + attn), axis=0, keepdims=True) * cnorm
        s_ref[...] = s.T
        for b2 in range(NBLK):
            blkmax = jnp.max(s[0, b2 * BLK:(b2 + 1) * BLK])
            flag_ref[b2] = (blkmax > 0.0).astype(jnp.int32)

    @pl.when(flag_ref[b] == 0)
    def _():
        o_ref[...] = jnp.zeros((BLK, D), jnp.float32)

    @pl.when(flag_ref[b] != 0)
    def _():
        cp = pltpu.make_async_copy(q_hbm.at[pl.ds(b * BLK, BLK)], qbuf, sem)
        cp.start()
        cp.wait()
        scol = s_ref[pl.ds(b * BLK, BLK), :]
        o_ref[...] = qbuf[...] * scol


def kernel(query, reference_points_cam, bev_mask):
    q = query[0]
    bm = jnp.transpose(bev_mask[:, 0], (2, 0, 1))
    rp = jnp.transpose(
        reference_points_cam[:, 0].reshape(C, N, 8), (2, 0, 1)
    )
    out = pl.pallas_call(
        _msda_body,
        grid=(NBLK,),
        in_specs=[
            pl.BlockSpec(memory_space=pl.ANY),
            pl.BlockSpec((4, C, N), lambda b: (0, 0, 0)),
            pl.BlockSpec((8, C, N), lambda b: (0, 0, 0)),
        ],
        out_specs=pl.BlockSpec((BLK, D), lambda b: (b, 0)),
        out_shape=jax.ShapeDtypeStruct((N, D), jnp.float32),
        scratch_shapes=[
            pltpu.VMEM((N, 1), jnp.float32),
            pltpu.SMEM((NBLK,), jnp.int32),
            pltpu.VMEM((BLK, D), jnp.float32),
            pltpu.SemaphoreType.DMA,
        ],
    )(q, bm, rp)
    return out[None]
